# baseline (speedup 1.0000x reference)
import functools

import jax
import jax.numpy as jnp
from jax.experimental import pallas as pl
from jax.experimental.pallas import tpu as pltpu


def _fused_moe_kernel(idx_ref, nw_ref, x_ref, g0_ref, g1_ref, u0_ref, u1_ref,
                      d0_ref, d1_ref, cw_ref, cb_ref, out_ref, halo_ref,
                      *, TB, D, F, KC, FB):
    i = pl.program_id(1)
    xb = x_ref[0]
    idx = idx_ref[0]
    nw = nw_ref[0]
    w = [jnp.sum(jnp.where(idx == e, nw, 0.0), axis=1, keepdims=True)
         for e in range(4)]

    halo = jnp.where(i > 0, halo_ref[8 - (KC - 1):, :], 0.0)
    xc = jnp.concatenate([halo, xb], axis=0)
    acc = jnp.zeros((TB, D), jnp.float32)
    for ce in range(2):
        c = jnp.zeros((TB, D), jnp.float32)
        for k in range(KC):
            c = c + xc[k:k + TB, :] * cw_ref[ce, k, :][None, :]
        c = c + cb_ref[ce, :][None, :]
        acc = acc + w[2 + ce] * jax.nn.silu(c)
    halo_ref[...] = xb[TB - 8:, :]
    out_ref[0] = acc

    xbb = xb.astype(jnp.bfloat16)
    for e, (g_ref, u_ref, d_ref) in enumerate(
            ((g0_ref, u0_ref, d0_ref), (g1_ref, u1_ref, d1_ref))):
        eacc = jnp.zeros((TB, D), jnp.float32)
        for fs in range(0, F, FB):
            g = jnp.dot(xbb, g_ref[0, :, fs:fs + FB],
                        preferred_element_type=jnp.float32)
            u = jnp.dot(xbb, u_ref[0, :, fs:fs + FB],
                        preferred_element_type=jnp.float32)
            h = (jax.nn.silu(g) * u).astype(jnp.bfloat16)
            eacc = eacc + jnp.dot(h, d_ref[0, fs:fs + FB, :],
                                  preferred_element_type=jnp.float32)
        out_ref[0] += w[e] * eacc


def kernel(x, top_k_indices, norm_weights, w_gate, w_up, w_down, conv_w, conv_b):
    B, S, D = x.shape
    F = w_gate.shape[2]
    KC = conv_w.shape[2]
    TOPK = top_k_indices.shape[2]
    TB = min(512, S)
    FB = min(512, F)
    nsb = S // TB

    wg = w_gate.astype(jnp.bfloat16)
    wu = w_up.astype(jnp.bfloat16)
    wd = w_down.astype(jnp.bfloat16)
    cwt = conv_w.transpose(0, 2, 1)

    kern = functools.partial(_fused_moe_kernel, TB=TB, D=D, F=F, KC=KC, FB=FB)
    wspec = pl.BlockSpec((1, D, F), lambda b, i: (0, 0, 0))
    wspec1 = pl.BlockSpec((1, D, F), lambda b, i: (1, 0, 0))
    dspec = pl.BlockSpec((1, F, D), lambda b, i: (0, 0, 0))
    dspec1 = pl.BlockSpec((1, F, D), lambda b, i: (1, 0, 0))
    out = pl.pallas_call(
        kern,
        grid=(B, nsb),
        in_specs=[
            pl.BlockSpec((1, TB, TOPK), lambda b, i: (b, i, 0)),
            pl.BlockSpec((1, TB, TOPK), lambda b, i: (b, i, 0)),
            pl.BlockSpec((1, TB, D), lambda b, i: (b, i, 0)),
            wspec, wspec1, wspec, wspec1, dspec, dspec1,
            pl.BlockSpec((2, KC, D), lambda b, i: (0, 0, 0)),
            pl.BlockSpec((2, D), lambda b, i: (0, 0)),
        ],
        out_specs=pl.BlockSpec((1, TB, D), lambda b, i: (b, i, 0)),
        out_shape=jax.ShapeDtypeStruct((B, S, D), jnp.float32),
        scratch_shapes=[pltpu.VMEM((8, D), jnp.float32)],
        compiler_params=pltpu.CompilerParams(
            dimension_semantics=("arbitrary", "arbitrary")),
    )(top_k_indices, norm_weights, x, wg, wg, wu, wu, wd, wd, cwt, conv_b)
    return out


# speedup vs baseline: 1.2090x; 1.2090x over previous
import functools

import jax
import jax.numpy as jnp
from jax.experimental import pallas as pl
from jax.experimental.pallas import tpu as pltpu


def _fused_moe_kernel(idx_ref, nw_ref, x_ref, g0_ref, g1_ref, u0_ref, u1_ref,
                      d0_ref, d1_ref, cw_ref, cb_ref, out_ref, halo_ref,
                      *, TB, D, F, KC, FB):
    i = pl.program_id(1)
    xb = x_ref[0]
    idx = idx_ref[0]
    nw = nw_ref[0]
    w = [jnp.sum(jnp.where(idx == e, nw, 0.0), axis=1, keepdims=True)
         for e in range(4)]

    xbb = xb.astype(jnp.bfloat16)
    halo = jnp.where(i > 0, halo_ref[8 - (KC - 1):, :], 0.0)
    xc = jnp.concatenate([halo.astype(jnp.bfloat16), xbb], axis=0)
    acc = jnp.zeros((TB, D), jnp.float32)
    for ce in range(2):
        c = jnp.zeros((TB, D), jnp.bfloat16)
        for k in range(KC):
            c = c + xc[k:k + TB, :] * cw_ref[ce, k, :][None, :]
        cf = c.astype(jnp.float32) + cb_ref[ce, :][None, :]
        acc = acc + w[2 + ce] * jax.nn.silu(cf)
    halo_ref[...] = xb[TB - 8:, :]

    for e, (g_ref, u_ref, d_ref) in enumerate(
            ((g0_ref, u0_ref, d0_ref), (g1_ref, u1_ref, d1_ref))):
        eacc = jnp.zeros((TB, D), jnp.float32)
        for fs in range(0, F, FB):
            g = jnp.dot(xbb, g_ref[0, :, fs:fs + FB],
                        preferred_element_type=jnp.float32)
            u = jnp.dot(xbb, u_ref[0, :, fs:fs + FB],
                        preferred_element_type=jnp.float32)
            h = (jax.nn.silu(g) * u).astype(jnp.bfloat16)
            eacc = eacc + jnp.dot(h, d_ref[0, fs:fs + FB, :],
                                  preferred_element_type=jnp.float32)
        acc = acc + w[e] * eacc
    out_ref[0] = acc


def kernel(x, top_k_indices, norm_weights, w_gate, w_up, w_down, conv_w, conv_b):
    B, S, D = x.shape
    F = w_gate.shape[2]
    KC = conv_w.shape[2]
    TOPK = top_k_indices.shape[2]
    TB = min(512, S)
    FB = min(512, F)
    nsb = S // TB

    wg = w_gate.astype(jnp.bfloat16)
    wu = w_up.astype(jnp.bfloat16)
    wd = w_down.astype(jnp.bfloat16)
    cwt = conv_w.transpose(0, 2, 1).astype(jnp.bfloat16)

    kern = functools.partial(_fused_moe_kernel, TB=TB, D=D, F=F, KC=KC, FB=FB)
    wspec = pl.BlockSpec((1, D, F), lambda b, i: (0, 0, 0))
    wspec1 = pl.BlockSpec((1, D, F), lambda b, i: (1, 0, 0))
    dspec = pl.BlockSpec((1, F, D), lambda b, i: (0, 0, 0))
    dspec1 = pl.BlockSpec((1, F, D), lambda b, i: (1, 0, 0))
    out = pl.pallas_call(
        kern,
        grid=(B, nsb),
        in_specs=[
            pl.BlockSpec((1, TB, TOPK), lambda b, i: (b, i, 0)),
            pl.BlockSpec((1, TB, TOPK), lambda b, i: (b, i, 0)),
            pl.BlockSpec((1, TB, D), lambda b, i: (b, i, 0)),
            wspec, wspec1, wspec, wspec1, dspec, dspec1,
            pl.BlockSpec((2, KC, D), lambda b, i: (0, 0, 0)),
            pl.BlockSpec((2, D), lambda b, i: (0, 0)),
        ],
        out_specs=pl.BlockSpec((1, TB, D), lambda b, i: (b, i, 0)),
        out_shape=jax.ShapeDtypeStruct((B, S, D), jnp.float32),
        scratch_shapes=[pltpu.VMEM((8, D), jnp.float32)],
        compiler_params=pltpu.CompilerParams(
            dimension_semantics=("arbitrary", "arbitrary")),
    )(top_k_indices, norm_weights, x, wg, wg, wu, wu, wd, wd, cwt, conv_b)
    return out


# speedup vs baseline: 1.2122x; 1.0027x over previous
import functools

import jax
import jax.numpy as jnp
from jax.experimental import pallas as pl
from jax.experimental.pallas import tpu as pltpu


def _fused_moe_kernel(idx_ref, nw_ref, x_ref, g0_ref, g1_ref, u0_ref, u1_ref,
                      d0_ref, d1_ref, cw_ref, cb_ref, out_ref, halo_ref,
                      *, TB, D, F, KC, FB):
    i = pl.program_id(1)
    xb = x_ref[0]
    idx = idx_ref[0]
    nw = nw_ref[0]
    w = [jnp.sum(jnp.where(idx == e, nw, 0.0), axis=1, keepdims=True)
         for e in range(4)]

    xbb = xb.astype(jnp.bfloat16)
    halo = jnp.where(i > 0, halo_ref[8 - (KC - 1):, :], 0.0)
    xc = jnp.concatenate([halo.astype(jnp.bfloat16), xbb], axis=0)
    acc = jnp.zeros((TB, D), jnp.float32)
    for ce in range(2):
        c = jnp.zeros((TB, D), jnp.bfloat16)
        for k in range(KC):
            c = c + xc[k:k + TB, :] * cw_ref[ce, k, :][None, :]
        cf = c.astype(jnp.float32) + cb_ref[ce, :][None, :]
        acc = acc + w[2 + ce] * jax.nn.silu(cf)
    halo_ref[...] = xb[TB - 8:, :]

    for e, (g_ref, u_ref, d_ref) in enumerate(
            ((g0_ref, u0_ref, d0_ref), (g1_ref, u1_ref, d1_ref))):
        eacc = jnp.zeros((TB, D), jnp.float32)
        for fs in range(0, F, FB):
            g = jnp.dot(xbb, g_ref[0, :, fs:fs + FB],
                        preferred_element_type=jnp.float32)
            u = jnp.dot(xbb, u_ref[0, :, fs:fs + FB],
                        preferred_element_type=jnp.float32)
            h = (jax.nn.silu(g) * u).astype(jnp.bfloat16)
            eacc = eacc + jnp.dot(h, d_ref[0, fs:fs + FB, :],
                                  preferred_element_type=jnp.float32)
        acc = acc + w[e] * eacc
    out_ref[0] = acc


def kernel(x, top_k_indices, norm_weights, w_gate, w_up, w_down, conv_w, conv_b):
    B, S, D = x.shape
    F = w_gate.shape[2]
    KC = conv_w.shape[2]
    TOPK = top_k_indices.shape[2]
    TB = min(512, S)
    FB = min(2048, F)
    nsb = S // TB

    wg = w_gate.astype(jnp.bfloat16)
    wu = w_up.astype(jnp.bfloat16)
    wd = w_down.astype(jnp.bfloat16)
    cwt = conv_w.transpose(0, 2, 1).astype(jnp.bfloat16)

    kern = functools.partial(_fused_moe_kernel, TB=TB, D=D, F=F, KC=KC, FB=FB)
    wspec = pl.BlockSpec((1, D, F), lambda b, i: (0, 0, 0))
    wspec1 = pl.BlockSpec((1, D, F), lambda b, i: (1, 0, 0))
    dspec = pl.BlockSpec((1, F, D), lambda b, i: (0, 0, 0))
    dspec1 = pl.BlockSpec((1, F, D), lambda b, i: (1, 0, 0))
    out = pl.pallas_call(
        kern,
        grid=(B, nsb),
        in_specs=[
            pl.BlockSpec((1, TB, TOPK), lambda b, i: (b, i, 0)),
            pl.BlockSpec((1, TB, TOPK), lambda b, i: (b, i, 0)),
            pl.BlockSpec((1, TB, D), lambda b, i: (b, i, 0)),
            wspec, wspec1, wspec, wspec1, dspec, dspec1,
            pl.BlockSpec((2, KC, D), lambda b, i: (0, 0, 0)),
            pl.BlockSpec((2, D), lambda b, i: (0, 0)),
        ],
        out_specs=pl.BlockSpec((1, TB, D), lambda b, i: (b, i, 0)),
        out_shape=jax.ShapeDtypeStruct((B, S, D), jnp.float32),
        scratch_shapes=[pltpu.VMEM((8, D), jnp.float32)],
        compiler_params=pltpu.CompilerParams(
            dimension_semantics=("arbitrary", "arbitrary")),
    )(top_k_indices, norm_weights, x, wg, wg, wu, wu, wd, wd, cwt, conv_b)
    return out
